# baseline (speedup 1.0000x reference)
import jax
import jax.numpy as jnp
from jax.experimental import pallas as pl
from jax.experimental.pallas import tpu as pltpu


def _fused_kernel(x_ref, w_ref, b_ref, a_ref, o_ref, h_ref):
    i = pl.program_id(0)

    @pl.when(i == 0)
    def _():
        h = jnp.dot(x_ref[...], w_ref[...],
                    preferred_element_type=jnp.float32)
        h_ref[...] = h.astype(jnp.bfloat16)

    @pl.when(i > 0)
    def _():
        a = a_ref[...].astype(jnp.bfloat16)
        o_ref[...] = jnp.dot(a, h_ref[...],
                             preferred_element_type=jnp.float32) + b_ref[...]


def kernel(features, adj, weight, bias):
    n, in_f = features.shape
    out_f = weight.shape[1]
    bm = 256
    bias2 = bias.reshape(1, out_f)

    def _shifted(i):
        return (jax.lax.max(i - 1, 0), 0)

    out = pl.pallas_call(
        _fused_kernel,
        grid=(1 + pl.cdiv(n, bm),),
        in_specs=[
            pl.BlockSpec((n, in_f), lambda i: (0, 0)),
            pl.BlockSpec((in_f, out_f), lambda i: (0, 0)),
            pl.BlockSpec((1, out_f), lambda i: (0, 0)),
            pl.BlockSpec((bm, n), _shifted),
        ],
        out_specs=pl.BlockSpec((bm, out_f), _shifted),
        out_shape=jax.ShapeDtypeStruct((n, out_f), jnp.float32),
        scratch_shapes=[pltpu.VMEM((n, out_f), jnp.bfloat16)],
    )(features, weight, bias2, adj)
    return out


# speedup vs baseline: 1.0150x; 1.0150x over previous
import jax
import jax.numpy as jnp
from jax.experimental import pallas as pl
from jax.experimental.pallas import tpu as pltpu


def _fused_kernel(x_ref, w_ref, b_ref, a_ref, o_ref, h_ref):
    i = pl.program_id(0)

    @pl.when(i == 0)
    def _():
        h_ref[...] = jnp.dot(x_ref[...], w_ref[...],
                             preferred_element_type=jnp.float32)

    o_ref[...] = jnp.dot(a_ref[...], h_ref[...],
                         preferred_element_type=jnp.float32) + b_ref[...]


def kernel(features, adj, weight, bias):
    n, in_f = features.shape
    out_f = weight.shape[1]
    bm = 256
    bias2 = bias.reshape(1, out_f)

    out = pl.pallas_call(
        _fused_kernel,
        grid=(pl.cdiv(n, bm),),
        in_specs=[
            pl.BlockSpec((n, in_f), lambda i: (0, 0)),
            pl.BlockSpec((in_f, out_f), lambda i: (0, 0)),
            pl.BlockSpec((1, out_f), lambda i: (0, 0)),
            pl.BlockSpec((bm, n), lambda i: (i, 0)),
        ],
        out_specs=pl.BlockSpec((bm, out_f), lambda i: (i, 0)),
        out_shape=jax.ShapeDtypeStruct((n, out_f), jnp.float32),
        scratch_shapes=[pltpu.VMEM((n, out_f), jnp.float32)],
    )(features, weight, bias2, adj)
    return out
